# Initial kernel scaffold; baseline (speedup 1.0000x reference)
import jax
import jax.numpy as jnp
from jax.experimental import pallas as pl


def kernel(x, x_, W_l, b_l, W_r, edge_index):
    raise NotImplementedError("write your pallas kernel here")


# speedup vs baseline: 5.5426x; 5.5426x over previous
import functools

import jax
import jax.numpy as jnp
from jax import lax
from jax.experimental import pallas as pl
from jax.experimental.pallas import tpu as pltpu
from jax.experimental.pallas import tpu_sc as plsc

N = 10000
D = 128
DA = 144
E = 320000
NC = 2
NS = 16
NW = NC * NS
EW = E // NW
CHUNK = 80
NCHUNK = EW // CHUNK
RPT = 624
TAIL = N - NS * RPT


def _sc_segsum(xa, src, dst, zeros_nda):
    mesh = plsc.VectorSubcoreMesh(core_axis_name="c", subcore_axis_name="s")

    @functools.partial(
        pl.kernel,
        out_type=jax.ShapeDtypeStruct((NC * N, DA), jnp.float32),
        mesh=mesh,
        compiler_params=pltpu.CompilerParams(use_tc_tiling_on_sc=False),
        scratch_types=[
            pltpu.VMEM((CHUNK,), jnp.int32),
            pltpu.VMEM((CHUNK,), jnp.int32),
            pltpu.VMEM((CHUNK, DA), jnp.float32),
            pltpu.VMEM_SHARED((N, DA), jnp.float32),
            pltpu.SemaphoreType.DMA,
        ],
    )
    def sc_kernel(xa_hbm, src_hbm, dst_hbm, znd_hbm, aggp_hbm,
                  idx_s, idx_d, rows, agg_sh, sem):
        c = lax.axis_index("c")
        s = lax.axis_index("s")
        r0 = s * RPT
        pltpu.sync_copy(znd_hbm.at[pl.ds(r0, RPT)], agg_sh.at[pl.ds(r0, RPT)])

        @pl.when(s == NS - 1)
        def _():
            t0 = NS * RPT
            pltpu.sync_copy(znd_hbm.at[pl.ds(t0, TAIL)],
                            agg_sh.at[pl.ds(t0, TAIL)])

        plsc.subcore_barrier()

        base = (c * NS + s) * EW

        def body(i, carry):
            off = pl.multiple_of(base + i * CHUNK, 8)
            pltpu.sync_copy(src_hbm.at[pl.ds(off, CHUNK)], idx_s)
            pltpu.sync_copy(dst_hbm.at[pl.ds(off, CHUNK)], idx_d)
            pltpu.async_copy(xa_hbm.at[idx_s], rows, sem).wait()
            pltpu.sync_copy(rows, agg_sh.at[idx_d], add=True)
            return carry

        lax.fori_loop(0, NCHUNK, body, 0)
        plsc.subcore_barrier()
        o0 = pl.multiple_of(c * N + r0, 8)
        pltpu.sync_copy(agg_sh.at[pl.ds(r0, RPT)], aggp_hbm.at[pl.ds(o0, RPT)])

        @pl.when(s == NS - 1)
        def _():
            t0 = NS * RPT
            ot = pl.multiple_of(c * N + t0, 8)
            pltpu.sync_copy(agg_sh.at[pl.ds(t0, TAIL)],
                            aggp_hbm.at[pl.ds(ot, TAIL)])

    return sc_kernel(xa, src, dst, zeros_nda)


BLK = 1000


def _tc_finish(aggp, x, x_, wlT, wrT, wsT, b2):
    def body(aggp_ref, x_ref, xp_ref, wl_ref, wr_ref, ws_ref,
             b_ref, out_ref, outp_ref):
        a = aggp_ref[0] + aggp_ref[1]
        agg = a[:, :D]
        cnt = jnp.maximum(a[:, D:D + 1], 1.0)
        mean = agg / cnt
        acc = jnp.dot(mean, wl_ref[...], preferred_element_type=jnp.float32)
        acc = acc + jnp.dot(x_ref[...], wr_ref[...],
                            preferred_element_type=jnp.float32)
        out_ref[...] = acc + b_ref[...]
        outp_ref[...] = jnp.dot(xp_ref[...], ws_ref[...],
                                preferred_element_type=jnp.float32) + b_ref[...]

    return pl.pallas_call(
        body,
        grid=(N // BLK,),
        in_specs=[
            pl.BlockSpec((NC, BLK, DA), lambda i: (0, i, 0)),
            pl.BlockSpec((BLK, D), lambda i: (i, 0)),
            pl.BlockSpec((BLK, D), lambda i: (i, 0)),
            pl.BlockSpec((D, D), lambda i: (0, 0)),
            pl.BlockSpec((D, D), lambda i: (0, 0)),
            pl.BlockSpec((D, D), lambda i: (0, 0)),
            pl.BlockSpec((1, D), lambda i: (0, 0)),
        ],
        out_specs=[
            pl.BlockSpec((BLK, D), lambda i: (i, 0)),
            pl.BlockSpec((BLK, D), lambda i: (i, 0)),
        ],
        out_shape=[
            jax.ShapeDtypeStruct((N, D), jnp.float32),
            jax.ShapeDtypeStruct((N, D), jnp.float32),
        ],
    )(aggp, x, x_, wlT, wrT, wsT, b2)


def kernel(x, x_, W_l, b_l, W_r, edge_index):
    src = edge_index[0]
    dst = edge_index[1]
    ones_col = jnp.ones((N, 1), jnp.float32)
    pad = jnp.zeros((N, DA - D - 1), jnp.float32)
    xa = jnp.concatenate([x, ones_col, pad], axis=1)
    zeros_nda = jnp.zeros((N, DA), jnp.float32)
    aggp = _sc_segsum(xa, src, dst, zeros_nda).reshape(NC, N, DA)
    wlT = W_l.T
    wrT = W_r.T
    wsT = (W_l + W_r).T
    b2 = b_l.reshape(1, D)
    out, out_ = _tc_finish(aggp, x, x_, wlT, wrT, wsT, b2)
    return (out, out_)


# speedup vs baseline: 10.3436x; 1.8662x over previous
import functools

import jax
import jax.numpy as jnp
from jax import lax
from jax.experimental import pallas as pl
from jax.experimental.pallas import tpu as pltpu
from jax.experimental.pallas import tpu_sc as plsc

N = 10000
D = 128
DA = 136
E = 320000
NC = 2
NS = 16
NW = NC * NS
EW = E // NW
CHUNK = 80
NCHUNK = EW // CHUNK
RPT = 624
TAIL = N - NS * RPT


def _sc_segsum(xa, src2, dst2, zeros_nda):
    mesh = plsc.VectorSubcoreMesh(core_axis_name="c", subcore_axis_name="s")

    @functools.partial(
        pl.kernel,
        out_type=jax.ShapeDtypeStruct((NC * N, DA), jnp.float32),
        mesh=mesh,
        compiler_params=pltpu.CompilerParams(use_tc_tiling_on_sc=False),
        scratch_types=[
            pltpu.VMEM((NCHUNK, CHUNK), jnp.int32),
            pltpu.VMEM((NCHUNK, CHUNK), jnp.int32),
            pltpu.VMEM((CHUNK, DA), jnp.float32),
            pltpu.VMEM((CHUNK, DA), jnp.float32),
            pltpu.VMEM_SHARED((N, DA), jnp.float32),
            pltpu.SemaphoreType.DMA,
            pltpu.SemaphoreType.DMA,
        ],
    )
    def sc_kernel(xa_hbm, src_hbm, dst_hbm, znd_hbm, aggp_hbm,
                  idx_s, idx_d, rows_a, rows_b, agg_sh, sem_a, sem_b):
        c = lax.axis_index("c")
        s = lax.axis_index("s")
        r0 = s * RPT
        pltpu.sync_copy(znd_hbm.at[pl.ds(r0, RPT)], agg_sh.at[pl.ds(r0, RPT)])

        @pl.when(s == NS - 1)
        def _():
            t0 = NS * RPT
            pltpu.sync_copy(znd_hbm.at[pl.ds(t0, TAIL)],
                            agg_sh.at[pl.ds(t0, TAIL)])

        wid = c * NS + s
        c0 = wid * NCHUNK
        pltpu.sync_copy(src_hbm.at[pl.ds(c0, NCHUNK)], idx_s)
        pltpu.sync_copy(dst_hbm.at[pl.ds(c0, NCHUNK)], idx_d)
        plsc.subcore_barrier()

        def gather_start(i, rows, sem):
            pltpu.async_copy(xa_hbm.at[idx_s.at[i]], rows, sem)

        def gather_wait(i, rows, sem):
            pltpu.make_async_copy(xa_hbm.at[idx_s.at[i]], rows, sem).wait()

        def scatter(i, rows):
            pltpu.sync_copy(rows, agg_sh.at[idx_d.at[i]], add=True)

        gather_start(0, rows_a, sem_a)

        def body(j, carry):
            i0 = 2 * j
            gather_start(i0 + 1, rows_b, sem_b)
            gather_wait(i0, rows_a, sem_a)
            scatter(i0, rows_a)
            gather_start(i0 + 2, rows_a, sem_a)
            gather_wait(i0 + 1, rows_b, sem_b)
            scatter(i0 + 1, rows_b)
            return carry

        lax.fori_loop(0, (NCHUNK - 1) // 2, body, 0)
        gather_wait(NCHUNK - 1, rows_a, sem_a)
        scatter(NCHUNK - 1, rows_a)
        plsc.subcore_barrier()
        o0 = pl.multiple_of(c * N + r0, 8)
        pltpu.sync_copy(agg_sh.at[pl.ds(r0, RPT)], aggp_hbm.at[pl.ds(o0, RPT)])

        @pl.when(s == NS - 1)
        def _():
            t0 = NS * RPT
            ot = pl.multiple_of(c * N + t0, 8)
            pltpu.sync_copy(agg_sh.at[pl.ds(t0, TAIL)],
                            aggp_hbm.at[pl.ds(ot, TAIL)])

    return sc_kernel(xa, src2, dst2, zeros_nda)


BLK = 1000


def _tc_finish(aggp, x, x_, wlT, wrT, wsT, b2):
    def body(aggp_ref, x_ref, xp_ref, wl_ref, wr_ref, ws_ref,
             b_ref, out_ref, outp_ref):
        a = aggp_ref[0] + aggp_ref[1]
        agg = a[:, :D]
        cnt = jnp.maximum(a[:, D:D + 1], 1.0)
        mean = agg / cnt
        acc = jnp.dot(mean, wl_ref[...], preferred_element_type=jnp.float32)
        acc = acc + jnp.dot(x_ref[...], wr_ref[...],
                            preferred_element_type=jnp.float32)
        out_ref[...] = acc + b_ref[...]
        outp_ref[...] = jnp.dot(xp_ref[...], ws_ref[...],
                                preferred_element_type=jnp.float32) + b_ref[...]

    return pl.pallas_call(
        body,
        grid=(N // BLK,),
        in_specs=[
            pl.BlockSpec((NC, BLK, DA), lambda i: (0, i, 0)),
            pl.BlockSpec((BLK, D), lambda i: (i, 0)),
            pl.BlockSpec((BLK, D), lambda i: (i, 0)),
            pl.BlockSpec((D, D), lambda i: (0, 0)),
            pl.BlockSpec((D, D), lambda i: (0, 0)),
            pl.BlockSpec((D, D), lambda i: (0, 0)),
            pl.BlockSpec((1, D), lambda i: (0, 0)),
        ],
        out_specs=[
            pl.BlockSpec((BLK, D), lambda i: (i, 0)),
            pl.BlockSpec((BLK, D), lambda i: (i, 0)),
        ],
        out_shape=[
            jax.ShapeDtypeStruct((N, D), jnp.float32),
            jax.ShapeDtypeStruct((N, D), jnp.float32),
        ],
    )(aggp, x, x_, wlT, wrT, wsT, b2)


def kernel(x, x_, W_l, b_l, W_r, edge_index):
    src2 = edge_index[0].reshape(E // CHUNK, CHUNK)
    dst2 = edge_index[1].reshape(E // CHUNK, CHUNK)
    ones_col = jnp.ones((N, 1), jnp.float32)
    pad = jnp.zeros((N, DA - D - 1), jnp.float32)
    xa = jnp.concatenate([x, ones_col, pad], axis=1)
    zeros_nda = jnp.zeros((N, DA), jnp.float32)
    aggp = _sc_segsum(xa, src2, dst2, zeros_nda).reshape(NC, N, DA)
    wlT = W_l.T
    wrT = W_r.T
    wsT = (W_l + W_r).T
    b2 = b_l.reshape(1, D)
    out, out_ = _tc_finish(aggp, x, x_, wlT, wrT, wsT, b2)
    return (out, out_)


# speedup vs baseline: 12.2357x; 1.1829x over previous
import functools

import jax
import jax.numpy as jnp
from jax import lax
from jax.experimental import pallas as pl
from jax.experimental.pallas import tpu as pltpu
from jax.experimental.pallas import tpu_sc as plsc

N = 10000
D = 128
E = 320000
NC = 2
NS = 16
NW = NC * NS
EW = E // NW
CHUNK = 80
NCHUNK = EW // CHUNK
RPT = 624
TAIL = N - NS * RPT


def _sc_segsum(x, src2, dst2, zeros_nd, zeros_n2, ones_c2):
    mesh = plsc.VectorSubcoreMesh(core_axis_name="c", subcore_axis_name="s")

    @functools.partial(
        pl.kernel,
        out_type=(
            jax.ShapeDtypeStruct((NC * N, D), jnp.float32),
            jax.ShapeDtypeStruct((NC * N, 8), jnp.float32),
        ),
        mesh=mesh,
        compiler_params=pltpu.CompilerParams(use_tc_tiling_on_sc=False),
        scratch_types=[
            pltpu.VMEM((NCHUNK, CHUNK), jnp.int32),
            pltpu.VMEM((NCHUNK, CHUNK), jnp.int32),
            pltpu.VMEM((CHUNK, D), jnp.float32),
            pltpu.VMEM((CHUNK, D), jnp.float32),
            pltpu.VMEM((CHUNK, 8), jnp.float32),
            pltpu.VMEM_SHARED((N, D), jnp.float32),
            pltpu.VMEM_SHARED((N, 8), jnp.float32),
            pltpu.SemaphoreType.DMA,
            pltpu.SemaphoreType.DMA,
        ],
    )
    def sc_kernel(x_hbm, src_hbm, dst_hbm, znd_hbm, zn2_hbm, ones_hbm,
                  aggp_hbm, cntp_hbm,
                  idx_s, idx_d, rows_a, rows_b, ones_v, agg_sh, cnt_sh,
                  sem_a, sem_b):
        c = lax.axis_index("c")
        s = lax.axis_index("s")
        r0 = s * RPT
        pltpu.sync_copy(znd_hbm.at[pl.ds(r0, RPT)], agg_sh.at[pl.ds(r0, RPT)])
        pltpu.sync_copy(zn2_hbm.at[pl.ds(r0, RPT)], cnt_sh.at[pl.ds(r0, RPT)])

        @pl.when(s == NS - 1)
        def _():
            t0 = NS * RPT
            pltpu.sync_copy(znd_hbm.at[pl.ds(t0, TAIL)],
                            agg_sh.at[pl.ds(t0, TAIL)])
            pltpu.sync_copy(zn2_hbm.at[pl.ds(t0, TAIL)],
                            cnt_sh.at[pl.ds(t0, TAIL)])

        wid = c * NS + s
        c0 = wid * NCHUNK
        pltpu.sync_copy(src_hbm.at[pl.ds(c0, NCHUNK)], idx_s)
        pltpu.sync_copy(dst_hbm.at[pl.ds(c0, NCHUNK)], idx_d)
        pltpu.sync_copy(ones_hbm, ones_v)
        plsc.subcore_barrier()

        def gather_start(i, rows, sem):
            pltpu.async_copy(x_hbm.at[idx_s.at[i]], rows, sem)

        def gather_wait(i, rows, sem):
            pltpu.make_async_copy(x_hbm.at[idx_s.at[i]], rows, sem).wait()

        def scatter(i, rows):
            pltpu.sync_copy(rows, agg_sh.at[idx_d.at[i]], add=True)
            pltpu.sync_copy(ones_v, cnt_sh.at[idx_d.at[i]], add=True)

        gather_start(0, rows_a, sem_a)

        def body(j, carry):
            i0 = 2 * j
            gather_start(i0 + 1, rows_b, sem_b)
            gather_wait(i0, rows_a, sem_a)
            scatter(i0, rows_a)
            gather_start(i0 + 2, rows_a, sem_a)
            gather_wait(i0 + 1, rows_b, sem_b)
            scatter(i0 + 1, rows_b)
            return carry

        lax.fori_loop(0, (NCHUNK - 1) // 2, body, 0)
        gather_wait(NCHUNK - 1, rows_a, sem_a)
        scatter(NCHUNK - 1, rows_a)
        plsc.subcore_barrier()
        o0 = pl.multiple_of(c * N + r0, 8)
        pltpu.sync_copy(agg_sh.at[pl.ds(r0, RPT)], aggp_hbm.at[pl.ds(o0, RPT)])
        pltpu.sync_copy(cnt_sh.at[pl.ds(r0, RPT)], cntp_hbm.at[pl.ds(o0, RPT)])

        @pl.when(s == NS - 1)
        def _():
            t0 = NS * RPT
            ot = pl.multiple_of(c * N + t0, 8)
            pltpu.sync_copy(agg_sh.at[pl.ds(t0, TAIL)],
                            aggp_hbm.at[pl.ds(ot, TAIL)])
            pltpu.sync_copy(cnt_sh.at[pl.ds(t0, TAIL)],
                            cntp_hbm.at[pl.ds(ot, TAIL)])

    return sc_kernel(x, src2, dst2, zeros_nd, zeros_n2, ones_c2)


BLK = 1000


def _tc_finish(aggp, cntp, x, x_, wlT, wrT, wsT, b2):
    def body(aggp_ref, cntp_ref, x_ref, xp_ref, wl_ref, wr_ref, ws_ref,
             b_ref, out_ref, outp_ref):
        agg = aggp_ref[0] + aggp_ref[1]
        cntb = cntp_ref[0] + cntp_ref[1]
        cnt = jnp.maximum(cntb[:, :1], 1.0)
        mean = agg / cnt
        acc = jnp.dot(mean, wl_ref[...], preferred_element_type=jnp.float32)
        acc = acc + jnp.dot(x_ref[...], wr_ref[...],
                            preferred_element_type=jnp.float32)
        out_ref[...] = acc + b_ref[...]
        outp_ref[...] = jnp.dot(xp_ref[...], ws_ref[...],
                                preferred_element_type=jnp.float32) + b_ref[...]

    return pl.pallas_call(
        body,
        grid=(N // BLK,),
        in_specs=[
            pl.BlockSpec((NC, BLK, D), lambda i: (0, i, 0)),
            pl.BlockSpec((NC, BLK, 8), lambda i: (0, i, 0)),
            pl.BlockSpec((BLK, D), lambda i: (i, 0)),
            pl.BlockSpec((BLK, D), lambda i: (i, 0)),
            pl.BlockSpec((D, D), lambda i: (0, 0)),
            pl.BlockSpec((D, D), lambda i: (0, 0)),
            pl.BlockSpec((D, D), lambda i: (0, 0)),
            pl.BlockSpec((1, D), lambda i: (0, 0)),
        ],
        out_specs=[
            pl.BlockSpec((BLK, D), lambda i: (i, 0)),
            pl.BlockSpec((BLK, D), lambda i: (i, 0)),
        ],
        out_shape=[
            jax.ShapeDtypeStruct((N, D), jnp.float32),
            jax.ShapeDtypeStruct((N, D), jnp.float32),
        ],
    )(aggp, cntp, x, x_, wlT, wrT, wsT, b2)


def kernel(x, x_, W_l, b_l, W_r, edge_index):
    src2 = edge_index[0].reshape(E // CHUNK, CHUNK)
    dst2 = edge_index[1].reshape(E // CHUNK, CHUNK)
    zeros_nd = jnp.zeros((N, D), jnp.float32)
    zeros_n2 = jnp.zeros((N, 8), jnp.float32)
    ones_c2 = jnp.ones((CHUNK, 8), jnp.float32)
    aggp, cntp = _sc_segsum(x, src2, dst2, zeros_nd, zeros_n2, ones_c2)
    aggp = aggp.reshape(NC, N, D)
    cntp = cntp.reshape(NC, N, 8)
    wlT = W_l.T
    wrT = W_r.T
    wsT = (W_l + W_r).T
    b2 = b_l.reshape(1, D)
    out, out_ = _tc_finish(aggp, cntp, x, x_, wlT, wrT, wsT, b2)
    return (out, out_)


# speedup vs baseline: 12.3878x; 1.0124x over previous
import functools

import jax
import jax.numpy as jnp
from jax import lax
from jax.experimental import pallas as pl
from jax.experimental.pallas import tpu as pltpu
from jax.experimental.pallas import tpu_sc as plsc

N = 10000
D = 128
E = 320000
NC = 2
NS = 16
NW = NC * NS
EW = E // NW
CHUNK = 80
NCHUNK = EW // CHUNK
RPT = 624
TAIL = N - NS * RPT


def _sc_segsum(x, src2, dst2, zrow, zc8, ones_c8):
    mesh = plsc.VectorSubcoreMesh(core_axis_name="c", subcore_axis_name="s")

    @functools.partial(
        pl.kernel,
        out_type=(
            jax.ShapeDtypeStruct((NC * N, D), jnp.float32),
            jax.ShapeDtypeStruct((NC * N, 8), jnp.float32),
        ),
        mesh=mesh,
        compiler_params=pltpu.CompilerParams(use_tc_tiling_on_sc=False),
        scratch_types=[
            pltpu.VMEM((NCHUNK, CHUNK), jnp.int32),
            pltpu.VMEM((NCHUNK, CHUNK), jnp.int32),
            pltpu.VMEM((CHUNK, D), jnp.float32),
            pltpu.VMEM((CHUNK, D), jnp.float32),
            pltpu.VMEM((CHUNK, 8), jnp.float32),
            pltpu.VMEM((CHUNK, 8), jnp.float32),
            pltpu.VMEM_SHARED((N, D), jnp.float32),
            pltpu.VMEM_SHARED((N, 8), jnp.float32),
            pltpu.SemaphoreType.DMA,
            pltpu.SemaphoreType.DMA,
        ],
    )
    def sc_kernel(x_hbm, src_hbm, dst_hbm, zrow_hbm, zc8_hbm, ones_hbm,
                  aggp_hbm, cntp_hbm,
                  idx_s, idx_d, rows_a, rows_b, ones_v, zc8_v, agg_sh, cnt_sh,
                  sem_a, sem_b):
        c = lax.axis_index("c")
        s = lax.axis_index("s")
        r0 = s * RPT
        pltpu.sync_copy(zrow_hbm, rows_a)
        pltpu.sync_copy(zc8_hbm, zc8_v)
        pltpu.sync_copy(ones_hbm, ones_v)
        for k in range(7):
            pltpu.sync_copy(rows_a, agg_sh.at[pl.ds(r0 + k * CHUNK, CHUNK)])
            pltpu.sync_copy(zc8_v, cnt_sh.at[pl.ds(r0 + k * CHUNK, CHUNK)])
        pltpu.sync_copy(rows_a.at[pl.ds(0, 64)],
                        agg_sh.at[pl.ds(r0 + 560, 64)])
        pltpu.sync_copy(zc8_v.at[pl.ds(0, 64)],
                        cnt_sh.at[pl.ds(r0 + 560, 64)])

        @pl.when(s == NS - 1)
        def _():
            t0 = NS * RPT
            pltpu.sync_copy(rows_a.at[pl.ds(0, TAIL)],
                            agg_sh.at[pl.ds(t0, TAIL)])
            pltpu.sync_copy(zc8_v.at[pl.ds(0, TAIL)],
                            cnt_sh.at[pl.ds(t0, TAIL)])

        wid = c * NS + s
        c0 = wid * NCHUNK
        pltpu.sync_copy(src_hbm.at[pl.ds(c0, NCHUNK)], idx_s)
        pltpu.sync_copy(dst_hbm.at[pl.ds(c0, NCHUNK)], idx_d)
        plsc.subcore_barrier()

        def gather_start(i, rows, sem):
            pltpu.async_copy(x_hbm.at[idx_s.at[i]], rows, sem)

        def gather_wait(i, rows, sem):
            pltpu.make_async_copy(x_hbm.at[idx_s.at[i]], rows, sem).wait()

        def scatter(i, rows):
            pltpu.sync_copy(rows, agg_sh.at[idx_d.at[i]], add=True)
            pltpu.sync_copy(ones_v, cnt_sh.at[idx_d.at[i]], add=True)

        gather_start(0, rows_a, sem_a)

        def body(j, carry):
            i0 = 2 * j
            gather_start(i0 + 1, rows_b, sem_b)
            gather_wait(i0, rows_a, sem_a)
            scatter(i0, rows_a)
            gather_start(i0 + 2, rows_a, sem_a)
            gather_wait(i0 + 1, rows_b, sem_b)
            scatter(i0 + 1, rows_b)
            return carry

        lax.fori_loop(0, (NCHUNK - 1) // 2, body, 0)
        gather_wait(NCHUNK - 1, rows_a, sem_a)
        scatter(NCHUNK - 1, rows_a)
        plsc.subcore_barrier()
        o0 = pl.multiple_of(c * N + r0, 8)
        pltpu.sync_copy(agg_sh.at[pl.ds(r0, RPT)], aggp_hbm.at[pl.ds(o0, RPT)])
        pltpu.sync_copy(cnt_sh.at[pl.ds(r0, RPT)], cntp_hbm.at[pl.ds(o0, RPT)])

        @pl.when(s == NS - 1)
        def _():
            t0 = NS * RPT
            ot = pl.multiple_of(c * N + t0, 8)
            pltpu.sync_copy(agg_sh.at[pl.ds(t0, TAIL)],
                            aggp_hbm.at[pl.ds(ot, TAIL)])
            pltpu.sync_copy(cnt_sh.at[pl.ds(t0, TAIL)],
                            cntp_hbm.at[pl.ds(ot, TAIL)])

    return sc_kernel(x, src2, dst2, zrow, zc8, ones_c8)


BLK = 1000


def _tc_indep(x, x_, wrT, wsT, b2):
    def body(x_ref, xp_ref, wr_ref, ws_ref, b_ref, xwr_ref, outp_ref):
        xwr_ref[...] = jnp.dot(x_ref[...], wr_ref[...],
                               preferred_element_type=jnp.float32) + b_ref[...]
        outp_ref[...] = jnp.dot(xp_ref[...], ws_ref[...],
                                preferred_element_type=jnp.float32) + b_ref[...]

    return pl.pallas_call(
        body,
        grid=(N // BLK,),
        in_specs=[
            pl.BlockSpec((BLK, D), lambda i: (i, 0)),
            pl.BlockSpec((BLK, D), lambda i: (i, 0)),
            pl.BlockSpec((D, D), lambda i: (0, 0)),
            pl.BlockSpec((D, D), lambda i: (0, 0)),
            pl.BlockSpec((1, D), lambda i: (0, 0)),
        ],
        out_specs=[
            pl.BlockSpec((BLK, D), lambda i: (i, 0)),
            pl.BlockSpec((BLK, D), lambda i: (i, 0)),
        ],
        out_shape=[
            jax.ShapeDtypeStruct((N, D), jnp.float32),
            jax.ShapeDtypeStruct((N, D), jnp.float32),
        ],
    )(x, x_, wrT, wsT, b2)


def _tc_finish(aggp, cntp, xwr, wlT):
    def body(aggp_ref, cntp_ref, xwr_ref, wl_ref, out_ref):
        agg = aggp_ref[0] + aggp_ref[1]
        cntb = cntp_ref[0] + cntp_ref[1]
        cnt = jnp.maximum(cntb[:, :1], 1.0)
        mean = agg / cnt
        out_ref[...] = jnp.dot(
            mean, wl_ref[...], preferred_element_type=jnp.float32
        ) + xwr_ref[...]

    return pl.pallas_call(
        body,
        grid=(N // BLK,),
        in_specs=[
            pl.BlockSpec((NC, BLK, D), lambda i: (0, i, 0)),
            pl.BlockSpec((NC, BLK, 8), lambda i: (0, i, 0)),
            pl.BlockSpec((BLK, D), lambda i: (i, 0)),
            pl.BlockSpec((D, D), lambda i: (0, 0)),
        ],
        out_specs=pl.BlockSpec((BLK, D), lambda i: (i, 0)),
        out_shape=jax.ShapeDtypeStruct((N, D), jnp.float32),
    )(aggp, cntp, xwr, wlT)


def kernel(x, x_, W_l, b_l, W_r, edge_index):
    src2 = edge_index[0].reshape(E // CHUNK, CHUNK)
    dst2 = edge_index[1].reshape(E // CHUNK, CHUNK)
    zrow = jnp.zeros((CHUNK, D), jnp.float32)
    zc8 = jnp.zeros((CHUNK, 8), jnp.float32)
    ones_c8 = jnp.ones((CHUNK, 8), jnp.float32)
    aggp, cntp = _sc_segsum(x, src2, dst2, zrow, zc8, ones_c8)
    aggp = aggp.reshape(NC, N, D)
    cntp = cntp.reshape(NC, N, 8)
    wlT = W_l.T
    wrT = W_r.T
    wsT = (W_l + W_r).T
    b2 = b_l.reshape(1, D)
    xwr, out_ = _tc_indep(x, x_, wrT, wsT, b2)
    out = _tc_finish(aggp, cntp, xwr, wlT)
    return (out, out_)


# speedup vs baseline: 13.0981x; 1.0573x over previous
import functools

import jax
import jax.numpy as jnp
from jax import lax
from jax.experimental import pallas as pl
from jax.experimental.pallas import tpu as pltpu
from jax.experimental.pallas import tpu_sc as plsc

N = 10000
D = 128
E = 320000
NC = 2
NS = 16
NW = NC * NS
EW = E // NW
CHUNK = 80
NCHUNK = EW // CHUNK
RPT = 624
TAIL = N - NS * RPT


def _sc_segsum(x, edge_index, zrow, zc8, ones_c8):
    mesh = plsc.VectorSubcoreMesh(core_axis_name="c", subcore_axis_name="s")

    @functools.partial(
        pl.kernel,
        out_type=(
            jax.ShapeDtypeStruct((NC * N, D), jnp.float32),
            jax.ShapeDtypeStruct((NC * N, 8), jnp.float32),
        ),
        mesh=mesh,
        compiler_params=pltpu.CompilerParams(use_tc_tiling_on_sc=False),
        scratch_types=[
            pltpu.VMEM((EW,), jnp.int32),
            pltpu.VMEM((EW,), jnp.int32),
            pltpu.VMEM((CHUNK, D), jnp.float32),
            pltpu.VMEM((CHUNK, D), jnp.float32),
            pltpu.VMEM((CHUNK, 8), jnp.float32),
            pltpu.VMEM((CHUNK, 8), jnp.float32),
            pltpu.VMEM_SHARED((N, D), jnp.float32),
            pltpu.VMEM_SHARED((N, 8), jnp.float32),
            pltpu.SemaphoreType.DMA,
            pltpu.SemaphoreType.DMA,
        ],
    )
    def sc_kernel(x_hbm, edge_hbm, zrow_hbm, zc8_hbm, ones_hbm,
                  aggp_hbm, cntp_hbm,
                  idx_s, idx_d, rows_a, rows_b, ones_v, zc8_v, agg_sh, cnt_sh,
                  sem_a, sem_b):
        c = lax.axis_index("c")
        s = lax.axis_index("s")
        r0 = s * RPT
        pltpu.sync_copy(zrow_hbm, rows_a)
        pltpu.sync_copy(zc8_hbm, zc8_v)
        pltpu.sync_copy(ones_hbm, ones_v)
        for k in range(7):
            pltpu.sync_copy(rows_a, agg_sh.at[pl.ds(r0 + k * CHUNK, CHUNK)])
            pltpu.sync_copy(zc8_v, cnt_sh.at[pl.ds(r0 + k * CHUNK, CHUNK)])
        pltpu.sync_copy(rows_a.at[pl.ds(0, 64)],
                        agg_sh.at[pl.ds(r0 + 560, 64)])
        pltpu.sync_copy(zc8_v.at[pl.ds(0, 64)],
                        cnt_sh.at[pl.ds(r0 + 560, 64)])

        @pl.when(s == NS - 1)
        def _():
            t0 = NS * RPT
            pltpu.sync_copy(rows_a.at[pl.ds(0, TAIL)],
                            agg_sh.at[pl.ds(t0, TAIL)])
            pltpu.sync_copy(zc8_v.at[pl.ds(0, TAIL)],
                            cnt_sh.at[pl.ds(t0, TAIL)])

        wid = c * NS + s
        e0 = pl.multiple_of(wid * EW, 8)
        pltpu.sync_copy(edge_hbm.at[0, pl.ds(e0, EW)], idx_s)
        pltpu.sync_copy(edge_hbm.at[1, pl.ds(e0, EW)], idx_d)
        plsc.subcore_barrier()

        def gather_start(i, rows, sem):
            pltpu.async_copy(
                x_hbm.at[idx_s.at[pl.ds(i * CHUNK, CHUNK)]], rows, sem)

        def gather_wait(i, rows, sem):
            pltpu.make_async_copy(
                x_hbm.at[idx_s.at[pl.ds(i * CHUNK, CHUNK)]], rows, sem).wait()

        def scatter(i, rows):
            dsl = idx_d.at[pl.ds(i * CHUNK, CHUNK)]
            pltpu.sync_copy(rows, agg_sh.at[dsl], add=True)
            pltpu.sync_copy(ones_v, cnt_sh.at[dsl], add=True)

        gather_start(0, rows_a, sem_a)

        def body(j, carry):
            i0 = 2 * j
            gather_start(i0 + 1, rows_b, sem_b)
            gather_wait(i0, rows_a, sem_a)
            scatter(i0, rows_a)
            gather_start(i0 + 2, rows_a, sem_a)
            gather_wait(i0 + 1, rows_b, sem_b)
            scatter(i0 + 1, rows_b)
            return carry

        lax.fori_loop(0, (NCHUNK - 1) // 2, body, 0)
        gather_wait(NCHUNK - 1, rows_a, sem_a)
        scatter(NCHUNK - 1, rows_a)
        plsc.subcore_barrier()
        o0 = pl.multiple_of(c * N + r0, 8)
        pltpu.sync_copy(agg_sh.at[pl.ds(r0, RPT)], aggp_hbm.at[pl.ds(o0, RPT)])
        pltpu.sync_copy(cnt_sh.at[pl.ds(r0, RPT)], cntp_hbm.at[pl.ds(o0, RPT)])

        @pl.when(s == NS - 1)
        def _():
            t0 = NS * RPT
            ot = pl.multiple_of(c * N + t0, 8)
            pltpu.sync_copy(agg_sh.at[pl.ds(t0, TAIL)],
                            aggp_hbm.at[pl.ds(ot, TAIL)])
            pltpu.sync_copy(cnt_sh.at[pl.ds(t0, TAIL)],
                            cntp_hbm.at[pl.ds(ot, TAIL)])

    return sc_kernel(x, edge_index, zrow, zc8, ones_c8)


BLK = 1000


def _tc_indep(x, x_, wrT, wsT, b2):
    def body(x_ref, xp_ref, wr_ref, ws_ref, b_ref, xwr_ref, outp_ref):
        xwr_ref[...] = jnp.dot(x_ref[...], wr_ref[...],
                               preferred_element_type=jnp.float32) + b_ref[...]
        outp_ref[...] = jnp.dot(xp_ref[...], ws_ref[...],
                                preferred_element_type=jnp.float32) + b_ref[...]

    return pl.pallas_call(
        body,
        grid=(N // BLK,),
        in_specs=[
            pl.BlockSpec((BLK, D), lambda i: (i, 0)),
            pl.BlockSpec((BLK, D), lambda i: (i, 0)),
            pl.BlockSpec((D, D), lambda i: (0, 0)),
            pl.BlockSpec((D, D), lambda i: (0, 0)),
            pl.BlockSpec((1, D), lambda i: (0, 0)),
        ],
        out_specs=[
            pl.BlockSpec((BLK, D), lambda i: (i, 0)),
            pl.BlockSpec((BLK, D), lambda i: (i, 0)),
        ],
        out_shape=[
            jax.ShapeDtypeStruct((N, D), jnp.float32),
            jax.ShapeDtypeStruct((N, D), jnp.float32),
        ],
    )(x, x_, wrT, wsT, b2)


def _tc_finish(aggp, cntp, xwr, wlT):
    def body(a0_ref, a1_ref, c0_ref, c1_ref, xwr_ref, wl_ref, out_ref):
        agg = a0_ref[...] + a1_ref[...]
        cntb = c0_ref[...] + c1_ref[...]
        cnt = jnp.maximum(cntb[:, :1], 1.0)
        mean = agg / cnt
        out_ref[...] = jnp.dot(
            mean, wl_ref[...], preferred_element_type=jnp.float32
        ) + xwr_ref[...]

    nb = N // BLK
    return pl.pallas_call(
        body,
        grid=(nb,),
        in_specs=[
            pl.BlockSpec((BLK, D), lambda i: (i, 0)),
            pl.BlockSpec((BLK, D), lambda i, _nb=nb: (_nb + i, 0)),
            pl.BlockSpec((BLK, 8), lambda i: (i, 0)),
            pl.BlockSpec((BLK, 8), lambda i, _nb=nb: (_nb + i, 0)),
            pl.BlockSpec((BLK, D), lambda i: (i, 0)),
            pl.BlockSpec((D, D), lambda i: (0, 0)),
        ],
        out_specs=pl.BlockSpec((BLK, D), lambda i: (i, 0)),
        out_shape=jax.ShapeDtypeStruct((N, D), jnp.float32),
    )(aggp, aggp, cntp, cntp, xwr, wlT)


def kernel(x, x_, W_l, b_l, W_r, edge_index):
    zrow = jnp.zeros((CHUNK, D), jnp.float32)
    zc8 = jnp.zeros((CHUNK, 8), jnp.float32)
    ones_c8 = jnp.ones((CHUNK, 8), jnp.float32)
    aggp, cntp = _sc_segsum(x, edge_index, zrow, zc8, ones_c8)
    wlT = W_l.T
    wrT = W_r.T
    wsT = (W_l + W_r).T
    b2 = b_l.reshape(1, D)
    xwr, out_ = _tc_indep(x, x_, wrT, wsT, b2)
    out = _tc_finish(aggp, cntp, xwr, wlT)
    return (out, out_)
